# baseline (speedup 1.0000x reference)
import functools

import jax
import jax.numpy as jnp
from jax import lax
from jax.experimental import pallas as pl
from jax.experimental.pallas import tpu as pltpu
from jax.experimental.pallas import tpu_sc as plsc

N = 10000
NP = 10240
E = 160000
ER = 1250
EPS = 1e-5
BLK = 2000
GRID = N // BLK

_MESH = plsc.VectorSubcoreMesh(
    core_axis_name="c", subcore_axis_name="s", num_cores=2, num_subcores=16)

F32 = jnp.float32


@functools.partial(
    pl.kernel,
    out_type=jax.ShapeDtypeStruct((32, NP), F32),
    mesh=_MESH,
    scratch_types=[
        pltpu.VMEM((NP,), F32),
        pltpu.VMEM((128,), jnp.int32),
        pltpu.SemaphoreType.DMA,
    ],
    compiler_params=pltpu.CompilerParams(needs_layout_passes=False),
)
def _sc_deg(dstm, zeros_in, outp, hist, drow, sem):
    c = lax.axis_index("c")
    s = lax.axis_index("s")
    wid = c * 16 + s
    pltpu.sync_copy(zeros_in, hist)
    nr = jnp.where(wid < 2, 40, 39)

    ones = jnp.full((16,), 1.0, F32)

    def eb(k, carry):
        row = wid + 32 * k
        pltpu.sync_copy(dstm.at[row], drow)
        for j in range(8):
            idx = drow[pl.ds(16 * j, 16)]
            plsc.addupdate_scatter(hist, [idx], ones)
        return carry

    lax.fori_loop(0, nr, eb, 0)
    pltpu.sync_copy(hist, outp.at[wid])


def _prop_pipeline(c, s, zs_by_core, accum, sbuf, dbuf, rows, sg, ss,
                   nfull):
    def g_start(i, b):
        @pl.when(c == 0)
        def _():
            pltpu.make_async_copy(
                zs_by_core[0].at[sbuf.at[i]], rows[b], sg[b]).start()

        @pl.when(c == 1)
        def _():
            pltpu.make_async_copy(
                zs_by_core[1].at[sbuf.at[i]], rows[b], sg[b]).start()

    def g_wait(b):
        @pl.when(c == 0)
        def _():
            pltpu.make_async_copy(
                zs_by_core[0].at[sbuf.at[0]], rows[b], sg[b]).wait()

        @pl.when(c == 1)
        def _():
            pltpu.make_async_copy(
                zs_by_core[1].at[sbuf.at[0]], rows[b], sg[b]).wait()

    def s_start(i, b):
        pltpu.make_async_copy(
            rows[b], accum.at[dbuf.at[i]], ss[b]).start(add=True)

    def s_wait(b):
        pltpu.make_async_copy(
            rows[b], accum.at[dbuf.at[0]], ss[b]).wait()

    g_start(0, 0)
    g_start(1, 1)

    def outer(k, carry):
        for b in range(2):
            g_wait(b)
            s_start(2 * k + b, b)
        for b in range(2):
            i = 2 * k + b
            s_wait(b)

            @pl.when(i + 2 < nfull)
            def _():
                g_start(i + 2, b)
        return carry

    lax.fori_loop(0, nfull // 2, outer, 0)


def _stage(srcm, dstm, sbuf, dbuf, base, n):
    base = pl.multiple_of(base, 8)
    pltpu.sync_copy(srcm.at[pl.ds(base, n), :], sbuf.at[pl.ds(0, n), :])
    pltpu.sync_copy(dstm.at[pl.ds(base, n), :], dbuf.at[pl.ds(0, n), :])


def _prop_chunk(c, s, zs0, zs1, t0, t1, srcm, dstm, accum, sbuf, dbuf,
                rows, sg, ss, zeros_in):
    rows0 = rows[0]
    pltpu.sync_copy(zeros_in, rows0)
    for j in range(5):
        pltpu.sync_copy(rows0, accum.at[pl.ds(s * 640 + j * 128, 128), :])
    plsc.subcore_barrier()
    baseA = jnp.where(s < 12, 80 * s, 960 + 72 * (s - 12))
    _stage(srcm, dstm, sbuf, dbuf, baseA, 40)
    _prop_pipeline(c, s, [zs0, zs1], accum, sbuf, dbuf, rows, sg, ss, 40)

    @pl.when(s < 12)
    def _():
        _stage(srcm, dstm, sbuf, dbuf, baseA + 40, 40)
        _prop_pipeline(c, s, [zs0, zs1], accum, sbuf, dbuf, rows, sg, ss, 40)

    @pl.when(s >= 12)
    def _():
        _stage(srcm, dstm, sbuf, dbuf, baseA + 40, 32)
        _prop_pipeline(c, s, [zs0, zs1], accum, sbuf, dbuf, rows, sg, ss, 32)

    @pl.when(s >= 14)
    def _():
        pltpu.sync_copy(srcm.at[1248 + (s - 14)], sbuf.at[0])
        pltpu.sync_copy(dstm.at[1248 + (s - 14)], dbuf.at[0])

        @pl.when(c == 0)
        def _():
            pltpu.async_copy(zs0.at[sbuf.at[0]], rows0, sg[0]).wait()

        @pl.when(c == 1)
        def _():
            pltpu.async_copy(zs1.at[sbuf.at[0]], rows0, sg[0]).wait()

        pltpu.sync_copy(rows0, accum.at[dbuf.at[0]], add=True)

    plsc.subcore_barrier()
    for j in range(5):
        pltpu.sync_copy(accum.at[pl.ds(s * 640 + j * 128, 128), :], rows0)

        @pl.when(c == 0)
        def _():
            pltpu.sync_copy(rows0, t0.at[pl.ds(s * 640 + j * 128, 128), :])

        @pl.when(c == 1)
        def _():
            pltpu.sync_copy(rows0, t1.at[pl.ds(s * 640 + j * 128, 128), :])


def _make_prop(nchunks):
    @functools.partial(
        pl.kernel,
        out_type=tuple(jax.ShapeDtypeStruct((NP, 128), F32)
                       for _ in range(2 * nchunks)),
        mesh=_MESH,
        scratch_types=[
            pltpu.VMEM_SHARED((NP, 128), F32),
            pltpu.VMEM((40, 128), jnp.int32),
            pltpu.VMEM((40, 128), jnp.int32),
            pltpu.VMEM((128, 128), F32),
            pltpu.VMEM((128, 128), F32),
            pltpu.SemaphoreType.DMA,
            pltpu.SemaphoreType.DMA,
            pltpu.SemaphoreType.DMA,
            pltpu.SemaphoreType.DMA,
        ],
    )
    def prop(*refs):
        zs = refs[:2 * nchunks]
        srcm, dstm, zeros_in = refs[2 * nchunks:2 * nchunks + 3]
        ts = refs[2 * nchunks + 3:4 * nchunks + 3]
        accum, sbuf, dbuf, r0, r1, sg0, sg1, ss0, ss1 = \
            refs[4 * nchunks + 3:]
        c = lax.axis_index("c")
        s = lax.axis_index("s")
        for ch in range(nchunks):
            _prop_chunk(c, s, zs[2 * ch], zs[2 * ch + 1],
                        ts[2 * ch], ts[2 * ch + 1], srcm, dstm,
                        accum, sbuf, dbuf, [r0, r1],
                        [sg0, sg1], [ss0, ss1], zeros_in)

    return prop


_sc_prop128 = _make_prop(1)
_sc_prop128x2 = _make_prop(2)


@functools.partial(
    pl.kernel,
    out_type=jax.ShapeDtypeStruct((2, NP, 128), F32),
    mesh=_MESH,
    scratch_types=[
        pltpu.VMEM_SHARED((NP, 128), F32),
        pltpu.VMEM((40, 128), jnp.int32),
        pltpu.VMEM((40, 128), jnp.int32),
        pltpu.VMEM((128, 128), F32),
        pltpu.VMEM((128, 128), F32),
        pltpu.SemaphoreType.DMA,
        pltpu.SemaphoreType.DMA,
        pltpu.SemaphoreType.DMA,
        pltpu.SemaphoreType.DMA,
    ],
)
def _sc_prop128_split(zsp, srcm, dstm, zeros_in, outp,
                      accum, sbuf, dbuf, rows0, rows1, sg0, sg1, ss0, ss1):
    c = lax.axis_index("c")
    s = lax.axis_index("s")
    rows = [rows0, rows1]
    pltpu.sync_copy(zeros_in, rows0)
    for j in range(5):
        pltpu.sync_copy(rows0, accum.at[pl.ds(s * 640 + j * 128, 128), :])
    plsc.subcore_barrier()
    baseA = 624 * c + jnp.where(s < 14, 40 * s, 560 + 32 * (s - 14))

    @pl.when(s < 14)
    def _():
        _stage(srcm, dstm, sbuf, dbuf, baseA, 40)
        _prop_pipeline(c, s, [zsp, zsp], accum, sbuf, dbuf, rows,
                       [sg0, sg1], [ss0, ss1], 40)

    @pl.when(s >= 14)
    def _():
        _stage(srcm, dstm, sbuf, dbuf, baseA, 32)
        _prop_pipeline(c, s, [zsp, zsp], accum, sbuf, dbuf, rows,
                       [sg0, sg1], [ss0, ss1], 32)

    @pl.when(s == 0)
    def _():
        pltpu.sync_copy(srcm.at[1248 + c], sbuf.at[0])
        pltpu.sync_copy(dstm.at[1248 + c], dbuf.at[0])
        pltpu.async_copy(zsp.at[sbuf.at[0]], rows0, sg0).wait()
        pltpu.sync_copy(rows0, accum.at[dbuf.at[0]], add=True)

    plsc.subcore_barrier()
    for j in range(5):
        pltpu.sync_copy(accum.at[pl.ds(s * 640 + j * 128, 128), :], rows0)
        pltpu.sync_copy(rows0, outp.at[c, pl.ds(s * 640 + j * 128, 128), :])


def _degred_body(degp, deg_ref):
    acc = 1.0 + degp[0]
    for w in range(1, 32):
        acc = acc + degp[w]
    deg_ref[...] = acc


def _pre_body(deg, x, dinv, za, zb):
    dv = lax.rsqrt(deg[...])
    dinv[...] = dv
    zs = x[...] * dv
    za[...] = zs[:, :128]
    zb[...] = zs[:, 128:]


def _l1_body(t1a, t1b, za, zb, dinv, w, b, y_ref, sums):
    i = pl.program_id(0)
    u = dinv[...] * jnp.concatenate(
        [t1a[...] + za[...], t1b[...] + zb[...]], axis=1)
    y = lax.dot_general(u, w[...], (((1,), (0,)), ((), ())),
                        preferred_element_type=F32) + b[...]
    y_ref[...] = y

    @pl.when(i == 0)
    def _():
        sums[...] = jnp.zeros_like(sums)

    sums[...] += jnp.concatenate(
        [jnp.sum(y, axis=0, keepdims=True),
         jnp.sum(y * y, axis=0, keepdims=True)], axis=1)


def _bn_mm_body(y, sums, g, be, w, dinv, z0, z1, z2, z3):
    mu = sums[0:1, :512] * (1.0 / N)
    var = sums[0:1, 512:] * (1.0 / N) - mu * mu
    h = jnp.maximum((y[...] - mu) * lax.rsqrt(var + EPS) * g[...] + be[...],
                    0.0)
    z = lax.dot_general(h, w[...], (((1,), (0,)), ((), ())),
                        preferred_element_type=F32) * dinv[...]
    z0[...] = z[:, 0:128]
    z1[...] = z[:, 128:256]
    z2[...] = z[:, 256:384]
    z3[...] = z[:, 384:512]


def _l2_body(t0, t1, t2, t3, z0, z1, z2, z3, dinv, b, v_ref, sums):
    i = pl.program_id(0)
    v = dinv[...] * jnp.concatenate(
        [t0[...] + z0[...], t1[...] + z1[...],
         t2[...] + z2[...], t3[...] + z3[...]], axis=1) + b[...]
    v_ref[...] = v

    @pl.when(i == 0)
    def _():
        sums[...] = jnp.zeros_like(sums)

    sums[...] += jnp.concatenate(
        [jnp.sum(v, axis=0, keepdims=True),
         jnp.sum(v * v, axis=0, keepdims=True)], axis=1)


def _bn_mm128_body(y, sums, g, be, w, dinv, z_ref):
    mu = sums[0:1, :512] * (1.0 / N)
    var = sums[0:1, 512:] * (1.0 / N) - mu * mu
    h = jnp.maximum((y[...] - mu) * lax.rsqrt(var + EPS) * g[...] + be[...],
                    0.0)
    z_ref[...] = lax.dot_general(h, w[...], (((1,), (0,)), ((), ())),
                                 preferred_element_type=F32) * dinv[...]


def _out_body(ta, tb, z, dinv, b, o_ref):
    o = dinv[...] * (ta[...] + tb[...] + z[...])
    o_ref[...] = o[:, :2] + b[...]


def _rb(w):
    return pl.BlockSpec((BLK, w), lambda i: (i, 0))


def _full(shape):
    return pl.BlockSpec(shape, lambda i: tuple(0 for _ in shape))


def kernel(x, edge_index, W1, b1, g1, be1, W2, b2, g2, be2, W3, b3):
    ei = edge_index.astype(jnp.int32)
    srcm = ei[0].reshape(ER, 128)
    dstm = ei[1].reshape(ER, 128)

    zerosNP = jnp.zeros((NP,), F32)
    zeros128 = jnp.zeros((128, 128), F32)

    degp = _sc_deg(dstm, zerosNP)

    deg1d = pl.pallas_call(
        _degred_body,
        grid=(1,),
        in_specs=[_full((32, NP))],
        out_specs=_full((NP,)),
        out_shape=jax.ShapeDtypeStruct((NP,), F32),
    )(degp)
    deg_col = deg1d.reshape(NP, 1)[:N]

    dinv, zs1a, zs1b = pl.pallas_call(
        _pre_body,
        grid=(GRID,),
        in_specs=[_rb(1), _rb(256)],
        out_specs=[_rb(1), _rb(128), _rb(128)],
        out_shape=[jax.ShapeDtypeStruct((N, 1), F32),
                   jax.ShapeDtypeStruct((N, 128), F32),
                   jax.ShapeDtypeStruct((N, 128), F32)],
    )(deg_col, x)

    t1a, t1b = _sc_prop128(zs1a, zs1b, srcm, dstm, zeros128)

    y1, sums1 = pl.pallas_call(
        _l1_body,
        grid=(GRID,),
        in_specs=[_rb(128), _rb(128), _rb(128), _rb(128), _rb(1),
                  _full((256, 512)), _full((1, 512))],
        out_specs=[_rb(512), _full((1, 1024))],
        out_shape=[jax.ShapeDtypeStruct((N, 512), F32),
                   jax.ShapeDtypeStruct((1, 1024), F32)],
    )(t1a, t1b, zs1a, zs1b, dinv, W1, b1.reshape(1, 512))

    zc = pl.pallas_call(
        _bn_mm_body,
        grid=(GRID,),
        in_specs=[_rb(512), _full((1, 1024)), _full((1, 512)),
                  _full((1, 512)), _full((512, 512)), _rb(1)],
        out_specs=[_rb(128)] * 4,
        out_shape=[jax.ShapeDtypeStruct((N, 128), F32)] * 4,
    )(y1, sums1, g1.reshape(1, 512), be1.reshape(1, 512), W2, dinv)

    t2c0, t2c1, t2c2, t2c3 = _sc_prop128x2(
        zc[0], zc[1], zc[2], zc[3], srcm, dstm, zeros128)

    v2, sums2 = pl.pallas_call(
        _l2_body,
        grid=(GRID,),
        in_specs=[_rb(128)] * 4 + [_rb(128)] * 4 + [_rb(1), _full((1, 512))],
        out_specs=[_rb(512), _full((1, 1024))],
        out_shape=[jax.ShapeDtypeStruct((N, 512), F32),
                   jax.ShapeDtypeStruct((1, 1024), F32)],
    )(t2c0, t2c1, t2c2, t2c3, zc[0], zc[1], zc[2], zc[3], dinv,
      b2.reshape(1, 512))

    W3p = jnp.pad(W3, ((0, 0), (0, 126)))
    zs3p = pl.pallas_call(
        _bn_mm128_body,
        grid=(GRID,),
        in_specs=[_rb(512), _full((1, 1024)), _full((1, 512)),
                  _full((1, 512)), _full((512, 128)), _rb(1)],
        out_specs=_rb(128),
        out_shape=jax.ShapeDtypeStruct((N, 128), F32),
    )(v2, sums2, g2.reshape(1, 512), be2.reshape(1, 512), W3p, dinv)

    t3p = _sc_prop128_split(zs3p, srcm, dstm, zeros128)

    out = pl.pallas_call(
        _out_body,
        grid=(GRID,),
        in_specs=[_rb(128), _rb(128), _rb(128), _rb(1), _full((1, 2))],
        out_specs=_rb(2),
        out_shape=jax.ShapeDtypeStruct((N, 2), F32),
    )(t3p[0], t3p[1], zs3p, dinv, b3.reshape(1, 2))
    return out


# speedup vs baseline: 1.2284x; 1.2284x over previous
import functools

import jax
import jax.numpy as jnp
from jax import lax
from jax.experimental import pallas as pl
from jax.experimental.pallas import tpu as pltpu
from jax.experimental.pallas import tpu_sc as plsc

N = 10000
NP = 10240
E = 160000
ER = 1250
EPS = 1e-5
BLK = 2000
GRID = N // BLK

_MESH = plsc.VectorSubcoreMesh(
    core_axis_name="c", subcore_axis_name="s", num_cores=2, num_subcores=16)

F32 = jnp.float32


@functools.partial(
    pl.kernel,
    out_type=jax.ShapeDtypeStruct((32, NP), F32),
    mesh=_MESH,
    scratch_types=[
        pltpu.VMEM((NP,), F32),
        pltpu.VMEM((128,), jnp.int32),
        pltpu.SemaphoreType.DMA,
    ],
    compiler_params=pltpu.CompilerParams(needs_layout_passes=False),
)
def _sc_deg(dstm, zeros_in, outp, hist, drow, sem):
    c = lax.axis_index("c")
    s = lax.axis_index("s")
    wid = c * 16 + s
    pltpu.sync_copy(zeros_in, hist)
    nr = jnp.where(wid < 2, 40, 39)

    ones = jnp.full((16,), 1.0, F32)

    def eb(k, carry):
        row = wid + 32 * k
        pltpu.sync_copy(dstm.at[row], drow)
        for j in range(8):
            idx = drow[pl.ds(16 * j, 16)]
            plsc.addupdate_scatter(hist, [idx], ones)
        return carry

    lax.fori_loop(0, nr, eb, 0)
    pltpu.sync_copy(hist, outp.at[wid])


def _prop_pipeline(c, s, zs_by_core, accum, sbuf, dbuf, rows, sg, ss,
                   nfull):
    def g_start(i, b):
        @pl.when(c == 0)
        def _():
            pltpu.make_async_copy(
                zs_by_core[0].at[sbuf.at[i]], rows[b], sg[b]).start()

        @pl.when(c == 1)
        def _():
            pltpu.make_async_copy(
                zs_by_core[1].at[sbuf.at[i]], rows[b], sg[b]).start()

    def g_wait(b):
        @pl.when(c == 0)
        def _():
            pltpu.make_async_copy(
                zs_by_core[0].at[sbuf.at[0]], rows[b], sg[b]).wait()

        @pl.when(c == 1)
        def _():
            pltpu.make_async_copy(
                zs_by_core[1].at[sbuf.at[0]], rows[b], sg[b]).wait()

    def s_start(i, b):
        pltpu.make_async_copy(
            rows[b], accum.at[dbuf.at[i]], ss[b]).start(add=True)

    def s_wait(b):
        pltpu.make_async_copy(
            rows[b], accum.at[dbuf.at[0]], ss[b]).wait()

    g_start(0, 0)
    g_start(1, 1)

    def outer(k, carry):
        for b in range(2):
            i = 2 * k + b
            g_wait(b)
            s_start(i, b)
            s_wait(b)

            @pl.when(i + 2 < nfull)
            def _():
                g_start(i + 2, b)
        return carry

    lax.fori_loop(0, nfull // 2, outer, 0)


def _stage(srcm, dstm, sbuf, dbuf, base, n):
    base = pl.multiple_of(base, 8)
    pltpu.sync_copy(srcm.at[pl.ds(base, n), :], sbuf.at[pl.ds(0, n), :])
    pltpu.sync_copy(dstm.at[pl.ds(base, n), :], dbuf.at[pl.ds(0, n), :])


def _prop_chunk(c, s, zs0, zs1, t0, t1, srcm, dstm, accum, sbuf, dbuf,
                rows, sg, ss, zeros_in):
    rows0 = rows[0]
    pltpu.sync_copy(zeros_in, rows0)
    for j in range(5):
        pltpu.sync_copy(rows0, accum.at[pl.ds(s * 640 + j * 128, 128), :])
    plsc.subcore_barrier()
    baseA = jnp.where(s < 12, 80 * s, 960 + 72 * (s - 12))
    _stage(srcm, dstm, sbuf, dbuf, baseA, 40)
    _prop_pipeline(c, s, [zs0, zs1], accum, sbuf, dbuf, rows, sg, ss, 40)

    @pl.when(s < 12)
    def _():
        _stage(srcm, dstm, sbuf, dbuf, baseA + 40, 40)
        _prop_pipeline(c, s, [zs0, zs1], accum, sbuf, dbuf, rows, sg, ss, 40)

    @pl.when(s >= 12)
    def _():
        _stage(srcm, dstm, sbuf, dbuf, baseA + 40, 32)
        _prop_pipeline(c, s, [zs0, zs1], accum, sbuf, dbuf, rows, sg, ss, 32)

    @pl.when(s >= 14)
    def _():
        pltpu.sync_copy(srcm.at[1248 + (s - 14)], sbuf.at[0])
        pltpu.sync_copy(dstm.at[1248 + (s - 14)], dbuf.at[0])

        @pl.when(c == 0)
        def _():
            pltpu.async_copy(zs0.at[sbuf.at[0]], rows0, sg[0]).wait()

        @pl.when(c == 1)
        def _():
            pltpu.async_copy(zs1.at[sbuf.at[0]], rows0, sg[0]).wait()

        pltpu.sync_copy(rows0, accum.at[dbuf.at[0]], add=True)

    plsc.subcore_barrier()
    for j in range(5):
        pltpu.sync_copy(accum.at[pl.ds(s * 640 + j * 128, 128), :], rows0)

        @pl.when(c == 0)
        def _():
            pltpu.sync_copy(rows0, t0.at[pl.ds(s * 640 + j * 128, 128), :])

        @pl.when(c == 1)
        def _():
            pltpu.sync_copy(rows0, t1.at[pl.ds(s * 640 + j * 128, 128), :])


def _make_prop(nchunks):
    @functools.partial(
        pl.kernel,
        out_type=tuple(jax.ShapeDtypeStruct((NP, 128), F32)
                       for _ in range(2 * nchunks)),
        mesh=_MESH,
        scratch_types=[
            pltpu.VMEM_SHARED((NP, 128), F32),
            pltpu.VMEM((40, 128), jnp.int32),
            pltpu.VMEM((40, 128), jnp.int32),
            pltpu.VMEM((128, 128), F32),
            pltpu.VMEM((128, 128), F32),
            pltpu.SemaphoreType.DMA,
            pltpu.SemaphoreType.DMA,
            pltpu.SemaphoreType.DMA,
            pltpu.SemaphoreType.DMA,
        ],
    )
    def prop(*refs):
        zs = refs[:2 * nchunks]
        srcm, dstm, zeros_in = refs[2 * nchunks:2 * nchunks + 3]
        ts = refs[2 * nchunks + 3:4 * nchunks + 3]
        accum, sbuf, dbuf, r0, r1, sg0, sg1, ss0, ss1 = \
            refs[4 * nchunks + 3:]
        c = lax.axis_index("c")
        s = lax.axis_index("s")
        for ch in range(nchunks):
            _prop_chunk(c, s, zs[2 * ch], zs[2 * ch + 1],
                        ts[2 * ch], ts[2 * ch + 1], srcm, dstm,
                        accum, sbuf, dbuf, [r0, r1],
                        [sg0, sg1], [ss0, ss1], zeros_in)

    return prop


_sc_prop128 = _make_prop(1)
_sc_prop128x2 = _make_prop(2)


@functools.partial(
    pl.kernel,
    out_type=jax.ShapeDtypeStruct((2, NP, 128), F32),
    mesh=_MESH,
    scratch_types=[
        pltpu.VMEM_SHARED((NP, 128), F32),
        pltpu.VMEM((40, 128), jnp.int32),
        pltpu.VMEM((40, 128), jnp.int32),
        pltpu.VMEM((128, 128), F32),
        pltpu.VMEM((128, 128), F32),
        pltpu.SemaphoreType.DMA,
        pltpu.SemaphoreType.DMA,
        pltpu.SemaphoreType.DMA,
        pltpu.SemaphoreType.DMA,
    ],
)
def _sc_prop128_split(zsp, srcm, dstm, zeros_in, outp,
                      accum, sbuf, dbuf, rows0, rows1, sg0, sg1, ss0, ss1):
    c = lax.axis_index("c")
    s = lax.axis_index("s")
    rows = [rows0, rows1]
    pltpu.sync_copy(zeros_in, rows0)
    for j in range(5):
        pltpu.sync_copy(rows0, accum.at[pl.ds(s * 640 + j * 128, 128), :])
    plsc.subcore_barrier()
    baseA = 624 * c + jnp.where(s < 14, 40 * s, 560 + 32 * (s - 14))

    @pl.when(s < 14)
    def _():
        _stage(srcm, dstm, sbuf, dbuf, baseA, 40)
        _prop_pipeline(c, s, [zsp, zsp], accum, sbuf, dbuf, rows,
                       [sg0, sg1], [ss0, ss1], 40)

    @pl.when(s >= 14)
    def _():
        _stage(srcm, dstm, sbuf, dbuf, baseA, 32)
        _prop_pipeline(c, s, [zsp, zsp], accum, sbuf, dbuf, rows,
                       [sg0, sg1], [ss0, ss1], 32)

    @pl.when(s == 0)
    def _():
        pltpu.sync_copy(srcm.at[1248 + c], sbuf.at[0])
        pltpu.sync_copy(dstm.at[1248 + c], dbuf.at[0])
        pltpu.async_copy(zsp.at[sbuf.at[0]], rows0, sg0).wait()
        pltpu.sync_copy(rows0, accum.at[dbuf.at[0]], add=True)

    plsc.subcore_barrier()
    for j in range(5):
        pltpu.sync_copy(accum.at[pl.ds(s * 640 + j * 128, 128), :], rows0)
        pltpu.sync_copy(rows0, outp.at[c, pl.ds(s * 640 + j * 128, 128), :])


def _degred_body(degp, deg_ref):
    acc = 1.0 + degp[0]
    for w in range(1, 32):
        acc = acc + degp[w]
    deg_ref[...] = acc


def _pre_body(deg, x, dinv, za, zb):
    dv = lax.rsqrt(deg[...])
    dinv[...] = dv
    zs = x[...] * dv
    za[...] = zs[:, :128]
    zb[...] = zs[:, 128:]


def _l1_body(t1a, t1b, za, zb, dinv, w, b, y_ref, sums):
    i = pl.program_id(0)
    u = dinv[...] * jnp.concatenate(
        [t1a[...] + za[...], t1b[...] + zb[...]], axis=1)
    y = lax.dot_general(u, w[...], (((1,), (0,)), ((), ())),
                        preferred_element_type=F32) + b[...]
    y_ref[...] = y

    @pl.when(i == 0)
    def _():
        sums[...] = jnp.zeros_like(sums)

    sums[...] += jnp.concatenate(
        [jnp.sum(y, axis=0, keepdims=True),
         jnp.sum(y * y, axis=0, keepdims=True)], axis=1)


def _bn_mm_body(y, sums, g, be, w, dinv, z0, z1, z2, z3):
    mu = sums[0:1, :512] * (1.0 / N)
    var = sums[0:1, 512:] * (1.0 / N) - mu * mu
    h = jnp.maximum((y[...] - mu) * lax.rsqrt(var + EPS) * g[...] + be[...],
                    0.0)
    z = lax.dot_general(h, w[...], (((1,), (0,)), ((), ())),
                        preferred_element_type=F32) * dinv[...]
    z0[...] = z[:, 0:128]
    z1[...] = z[:, 128:256]
    z2[...] = z[:, 256:384]
    z3[...] = z[:, 384:512]


def _l2_body(t0, t1, t2, t3, z0, z1, z2, z3, dinv, b, v_ref, sums):
    i = pl.program_id(0)
    v = dinv[...] * jnp.concatenate(
        [t0[...] + z0[...], t1[...] + z1[...],
         t2[...] + z2[...], t3[...] + z3[...]], axis=1) + b[...]
    v_ref[...] = v

    @pl.when(i == 0)
    def _():
        sums[...] = jnp.zeros_like(sums)

    sums[...] += jnp.concatenate(
        [jnp.sum(v, axis=0, keepdims=True),
         jnp.sum(v * v, axis=0, keepdims=True)], axis=1)


def _bn_mm128_body(y, sums, g, be, w, dinv, z_ref):
    mu = sums[0:1, :512] * (1.0 / N)
    var = sums[0:1, 512:] * (1.0 / N) - mu * mu
    h = jnp.maximum((y[...] - mu) * lax.rsqrt(var + EPS) * g[...] + be[...],
                    0.0)
    z_ref[...] = lax.dot_general(h, w[...], (((1,), (0,)), ((), ())),
                                 preferred_element_type=F32) * dinv[...]


def _out_body(ta, tb, z, dinv, b, o_ref):
    o = dinv[...] * (ta[...] + tb[...] + z[...])
    o_ref[...] = o[:, :2] + b[...]


def _rb(w):
    return pl.BlockSpec((BLK, w), lambda i: (i, 0))


def _full(shape):
    return pl.BlockSpec(shape, lambda i: tuple(0 for _ in shape))


def kernel(x, edge_index, W1, b1, g1, be1, W2, b2, g2, be2, W3, b3):
    ei = edge_index.astype(jnp.int32)
    srcm = ei[0].reshape(ER, 128)
    dstm = ei[1].reshape(ER, 128)

    zerosNP = jnp.zeros((NP,), F32)
    zeros128 = jnp.zeros((128, 128), F32)

    degp = _sc_deg(dstm, zerosNP)

    deg1d = pl.pallas_call(
        _degred_body,
        grid=(1,),
        in_specs=[_full((32, NP))],
        out_specs=_full((NP,)),
        out_shape=jax.ShapeDtypeStruct((NP,), F32),
    )(degp)
    deg_col = deg1d.reshape(NP, 1)[:N]

    dinv, zs1a, zs1b = pl.pallas_call(
        _pre_body,
        grid=(GRID,),
        in_specs=[_rb(1), _rb(256)],
        out_specs=[_rb(1), _rb(128), _rb(128)],
        out_shape=[jax.ShapeDtypeStruct((N, 1), F32),
                   jax.ShapeDtypeStruct((N, 128), F32),
                   jax.ShapeDtypeStruct((N, 128), F32)],
    )(deg_col, x)

    t1a, t1b = _sc_prop128(zs1a, zs1b, srcm, dstm, zeros128)

    y1, sums1 = pl.pallas_call(
        _l1_body,
        grid=(GRID,),
        in_specs=[_rb(128), _rb(128), _rb(128), _rb(128), _rb(1),
                  _full((256, 512)), _full((1, 512))],
        out_specs=[_rb(512), _full((1, 1024))],
        out_shape=[jax.ShapeDtypeStruct((N, 512), F32),
                   jax.ShapeDtypeStruct((1, 1024), F32)],
    )(t1a, t1b, zs1a, zs1b, dinv, W1, b1.reshape(1, 512))

    zc = pl.pallas_call(
        _bn_mm_body,
        grid=(GRID,),
        in_specs=[_rb(512), _full((1, 1024)), _full((1, 512)),
                  _full((1, 512)), _full((512, 512)), _rb(1)],
        out_specs=[_rb(128)] * 4,
        out_shape=[jax.ShapeDtypeStruct((N, 128), F32)] * 4,
    )(y1, sums1, g1.reshape(1, 512), be1.reshape(1, 512), W2, dinv)

    t2c0, t2c1, t2c2, t2c3 = _sc_prop128x2(
        zc[0], zc[1], zc[2], zc[3], srcm, dstm, zeros128)

    v2, sums2 = pl.pallas_call(
        _l2_body,
        grid=(GRID,),
        in_specs=[_rb(128)] * 4 + [_rb(128)] * 4 + [_rb(1), _full((1, 512))],
        out_specs=[_rb(512), _full((1, 1024))],
        out_shape=[jax.ShapeDtypeStruct((N, 512), F32),
                   jax.ShapeDtypeStruct((1, 1024), F32)],
    )(t2c0, t2c1, t2c2, t2c3, zc[0], zc[1], zc[2], zc[3], dinv,
      b2.reshape(1, 512))

    W3p = jnp.pad(W3, ((0, 0), (0, 126)))
    zs3p = pl.pallas_call(
        _bn_mm128_body,
        grid=(GRID,),
        in_specs=[_rb(512), _full((1, 1024)), _full((1, 512)),
                  _full((1, 512)), _full((512, 128)), _rb(1)],
        out_specs=_rb(128),
        out_shape=jax.ShapeDtypeStruct((N, 128), F32),
    )(v2, sums2, g2.reshape(1, 512), be2.reshape(1, 512), W3p, dinv)

    t3p = _sc_prop128_split(zs3p, srcm, dstm, zeros128)

    out = pl.pallas_call(
        _out_body,
        grid=(GRID,),
        in_specs=[_rb(128), _rb(128), _rb(128), _rb(1), _full((1, 2))],
        out_specs=_rb(2),
        out_shape=jax.ShapeDtypeStruct((N, 2), F32),
    )(t3p[0], t3p[1], zs3p, dinv, b3.reshape(1, 2))
    return out


# speedup vs baseline: 1.2304x; 1.0016x over previous
import functools

import jax
import jax.numpy as jnp
from jax import lax
from jax.experimental import pallas as pl
from jax.experimental.pallas import tpu as pltpu
from jax.experimental.pallas import tpu_sc as plsc

N = 10000
NP = 10240
E = 160000
ER = 1250
EPS = 1e-5
BLK = 2000
GRID = N // BLK

_MESH = plsc.VectorSubcoreMesh(
    core_axis_name="c", subcore_axis_name="s", num_cores=2, num_subcores=16)

F32 = jnp.float32


@functools.partial(
    pl.kernel,
    out_type=jax.ShapeDtypeStruct((32, NP), F32),
    mesh=_MESH,
    scratch_types=[
        pltpu.VMEM((NP,), F32),
        pltpu.VMEM((128,), jnp.int32),
        pltpu.SemaphoreType.DMA,
    ],
    compiler_params=pltpu.CompilerParams(needs_layout_passes=False),
)
def _sc_deg(dstm, zeros_in, outp, hist, drow, sem):
    c = lax.axis_index("c")
    s = lax.axis_index("s")
    wid = c * 16 + s
    pltpu.sync_copy(zeros_in, hist)
    nr = jnp.where(wid < 2, 40, 39)

    ones = jnp.full((16,), 1.0, F32)

    def eb(k, carry):
        row = wid + 32 * k
        pltpu.sync_copy(dstm.at[row], drow)
        for j in range(8):
            idx = drow[pl.ds(16 * j, 16)]
            plsc.addupdate_scatter(hist, [idx], ones)
        return carry

    lax.fori_loop(0, nr, eb, 0)
    pltpu.sync_copy(hist, outp.at[wid])


def _prop_pipeline(c, s, zs_by_core, accum, sbuf, dbuf, rows, sg, ss,
                   nfull):
    def g_start(i, b):
        @pl.when(c == 0)
        def _():
            for h in range(2):
                pltpu.make_async_copy(
                    zs_by_core[0].at[sbuf.at[i, pl.ds(64 * h, 64)]],
                    rows[b].at[pl.ds(64 * h, 64), :], sg[b]).start()

        @pl.when(c == 1)
        def _():
            for h in range(2):
                pltpu.make_async_copy(
                    zs_by_core[1].at[sbuf.at[i, pl.ds(64 * h, 64)]],
                    rows[b].at[pl.ds(64 * h, 64), :], sg[b]).start()

    def g_wait(b):
        @pl.when(c == 0)
        def _():
            pltpu.make_async_copy(
                zs_by_core[0].at[sbuf.at[0]], rows[b], sg[b]).wait()

        @pl.when(c == 1)
        def _():
            pltpu.make_async_copy(
                zs_by_core[1].at[sbuf.at[0]], rows[b], sg[b]).wait()

    def s_start(i, b):
        pltpu.make_async_copy(
            rows[b], accum.at[dbuf.at[i]], ss[b]).start(add=True)

    def s_wait(b):
        pltpu.make_async_copy(
            rows[b], accum.at[dbuf.at[0]], ss[b]).wait()

    g_start(0, 0)
    g_start(1, 1)

    def outer(k, carry):
        for b in range(2):
            i = 2 * k + b
            g_wait(b)
            s_start(i, b)
            s_wait(b)

            @pl.when(i + 2 < nfull)
            def _():
                g_start(i + 2, b)
        return carry

    lax.fori_loop(0, nfull // 2, outer, 0)


def _stage(srcm, dstm, sbuf, dbuf, base, n):
    base = pl.multiple_of(base, 8)
    pltpu.sync_copy(srcm.at[pl.ds(base, n), :], sbuf.at[pl.ds(0, n), :])
    pltpu.sync_copy(dstm.at[pl.ds(base, n), :], dbuf.at[pl.ds(0, n), :])


def _prop_chunk(c, s, zs0, zs1, t0, t1, srcm, dstm, accum, sbuf, dbuf,
                rows, sg, ss, zeros_in):
    rows0 = rows[0]
    pltpu.sync_copy(zeros_in, rows0)
    for j in range(5):
        pltpu.sync_copy(rows0, accum.at[pl.ds(s * 640 + j * 128, 128), :])
    plsc.subcore_barrier()
    baseA = jnp.where(s < 12, 80 * s, 960 + 72 * (s - 12))
    _stage(srcm, dstm, sbuf, dbuf, baseA, 40)
    _prop_pipeline(c, s, [zs0, zs1], accum, sbuf, dbuf, rows, sg, ss, 40)

    @pl.when(s < 12)
    def _():
        _stage(srcm, dstm, sbuf, dbuf, baseA + 40, 40)
        _prop_pipeline(c, s, [zs0, zs1], accum, sbuf, dbuf, rows, sg, ss, 40)

    @pl.when(s >= 12)
    def _():
        _stage(srcm, dstm, sbuf, dbuf, baseA + 40, 32)
        _prop_pipeline(c, s, [zs0, zs1], accum, sbuf, dbuf, rows, sg, ss, 32)

    @pl.when(s >= 14)
    def _():
        pltpu.sync_copy(srcm.at[1248 + (s - 14)], sbuf.at[0])
        pltpu.sync_copy(dstm.at[1248 + (s - 14)], dbuf.at[0])

        @pl.when(c == 0)
        def _():
            pltpu.async_copy(zs0.at[sbuf.at[0]], rows0, sg[0]).wait()

        @pl.when(c == 1)
        def _():
            pltpu.async_copy(zs1.at[sbuf.at[0]], rows0, sg[0]).wait()

        pltpu.sync_copy(rows0, accum.at[dbuf.at[0]], add=True)

    plsc.subcore_barrier()
    for j in range(5):
        pltpu.sync_copy(accum.at[pl.ds(s * 640 + j * 128, 128), :], rows0)

        @pl.when(c == 0)
        def _():
            pltpu.sync_copy(rows0, t0.at[pl.ds(s * 640 + j * 128, 128), :])

        @pl.when(c == 1)
        def _():
            pltpu.sync_copy(rows0, t1.at[pl.ds(s * 640 + j * 128, 128), :])


def _make_prop(nchunks):
    @functools.partial(
        pl.kernel,
        out_type=tuple(jax.ShapeDtypeStruct((NP, 128), F32)
                       for _ in range(2 * nchunks)),
        mesh=_MESH,
        scratch_types=[
            pltpu.VMEM_SHARED((NP, 128), F32),
            pltpu.VMEM((40, 128), jnp.int32),
            pltpu.VMEM((40, 128), jnp.int32),
            pltpu.VMEM((128, 128), F32),
            pltpu.VMEM((128, 128), F32),
            pltpu.SemaphoreType.DMA,
            pltpu.SemaphoreType.DMA,
            pltpu.SemaphoreType.DMA,
            pltpu.SemaphoreType.DMA,
        ],
    )
    def prop(*refs):
        zs = refs[:2 * nchunks]
        srcm, dstm, zeros_in = refs[2 * nchunks:2 * nchunks + 3]
        ts = refs[2 * nchunks + 3:4 * nchunks + 3]
        accum, sbuf, dbuf, r0, r1, sg0, sg1, ss0, ss1 = \
            refs[4 * nchunks + 3:]
        c = lax.axis_index("c")
        s = lax.axis_index("s")
        for ch in range(nchunks):
            _prop_chunk(c, s, zs[2 * ch], zs[2 * ch + 1],
                        ts[2 * ch], ts[2 * ch + 1], srcm, dstm,
                        accum, sbuf, dbuf, [r0, r1],
                        [sg0, sg1], [ss0, ss1], zeros_in)

    return prop


_sc_prop128 = _make_prop(1)
_sc_prop128x2 = _make_prop(2)


@functools.partial(
    pl.kernel,
    out_type=jax.ShapeDtypeStruct((2, NP, 128), F32),
    mesh=_MESH,
    scratch_types=[
        pltpu.VMEM_SHARED((NP, 128), F32),
        pltpu.VMEM((40, 128), jnp.int32),
        pltpu.VMEM((40, 128), jnp.int32),
        pltpu.VMEM((128, 128), F32),
        pltpu.VMEM((128, 128), F32),
        pltpu.SemaphoreType.DMA,
        pltpu.SemaphoreType.DMA,
        pltpu.SemaphoreType.DMA,
        pltpu.SemaphoreType.DMA,
    ],
)
def _sc_prop128_split(zsp, srcm, dstm, zeros_in, outp,
                      accum, sbuf, dbuf, rows0, rows1, sg0, sg1, ss0, ss1):
    c = lax.axis_index("c")
    s = lax.axis_index("s")
    rows = [rows0, rows1]
    pltpu.sync_copy(zeros_in, rows0)
    for j in range(5):
        pltpu.sync_copy(rows0, accum.at[pl.ds(s * 640 + j * 128, 128), :])
    plsc.subcore_barrier()
    baseA = 624 * c + jnp.where(s < 14, 40 * s, 560 + 32 * (s - 14))

    @pl.when(s < 14)
    def _():
        _stage(srcm, dstm, sbuf, dbuf, baseA, 40)
        _prop_pipeline(c, s, [zsp, zsp], accum, sbuf, dbuf, rows,
                       [sg0, sg1], [ss0, ss1], 40)

    @pl.when(s >= 14)
    def _():
        _stage(srcm, dstm, sbuf, dbuf, baseA, 32)
        _prop_pipeline(c, s, [zsp, zsp], accum, sbuf, dbuf, rows,
                       [sg0, sg1], [ss0, ss1], 32)

    @pl.when(s == 0)
    def _():
        pltpu.sync_copy(srcm.at[1248 + c], sbuf.at[0])
        pltpu.sync_copy(dstm.at[1248 + c], dbuf.at[0])
        pltpu.async_copy(zsp.at[sbuf.at[0]], rows0, sg0).wait()
        pltpu.sync_copy(rows0, accum.at[dbuf.at[0]], add=True)

    plsc.subcore_barrier()
    for j in range(5):
        pltpu.sync_copy(accum.at[pl.ds(s * 640 + j * 128, 128), :], rows0)
        pltpu.sync_copy(rows0, outp.at[c, pl.ds(s * 640 + j * 128, 128), :])


def _degred_body(degp, deg_ref):
    acc = 1.0 + degp[0]
    for w in range(1, 32):
        acc = acc + degp[w]
    deg_ref[...] = acc


def _pre_body(deg, x, dinv, za, zb):
    dv = lax.rsqrt(deg[...])
    dinv[...] = dv
    zs = x[...] * dv
    za[...] = zs[:, :128]
    zb[...] = zs[:, 128:]


def _l1_body(t1a, t1b, za, zb, dinv, w, b, y_ref, sums):
    i = pl.program_id(0)
    u = dinv[...] * jnp.concatenate(
        [t1a[...] + za[...], t1b[...] + zb[...]], axis=1)
    y = lax.dot_general(u, w[...], (((1,), (0,)), ((), ())),
                        preferred_element_type=F32) + b[...]
    y_ref[...] = y

    @pl.when(i == 0)
    def _():
        sums[...] = jnp.zeros_like(sums)

    sums[...] += jnp.concatenate(
        [jnp.sum(y, axis=0, keepdims=True),
         jnp.sum(y * y, axis=0, keepdims=True)], axis=1)


def _bn_mm_body(y, sums, g, be, w, dinv, z0, z1, z2, z3):
    mu = sums[0:1, :512] * (1.0 / N)
    var = sums[0:1, 512:] * (1.0 / N) - mu * mu
    h = jnp.maximum((y[...] - mu) * lax.rsqrt(var + EPS) * g[...] + be[...],
                    0.0)
    z = lax.dot_general(h, w[...], (((1,), (0,)), ((), ())),
                        preferred_element_type=F32) * dinv[...]
    z0[...] = z[:, 0:128]
    z1[...] = z[:, 128:256]
    z2[...] = z[:, 256:384]
    z3[...] = z[:, 384:512]


def _l2_body(t0, t1, t2, t3, z0, z1, z2, z3, dinv, b, v_ref, sums):
    i = pl.program_id(0)
    v = dinv[...] * jnp.concatenate(
        [t0[...] + z0[...], t1[...] + z1[...],
         t2[...] + z2[...], t3[...] + z3[...]], axis=1) + b[...]
    v_ref[...] = v

    @pl.when(i == 0)
    def _():
        sums[...] = jnp.zeros_like(sums)

    sums[...] += jnp.concatenate(
        [jnp.sum(v, axis=0, keepdims=True),
         jnp.sum(v * v, axis=0, keepdims=True)], axis=1)


def _bn_mm128_body(y, sums, g, be, w, dinv, z_ref):
    mu = sums[0:1, :512] * (1.0 / N)
    var = sums[0:1, 512:] * (1.0 / N) - mu * mu
    h = jnp.maximum((y[...] - mu) * lax.rsqrt(var + EPS) * g[...] + be[...],
                    0.0)
    z_ref[...] = lax.dot_general(h, w[...], (((1,), (0,)), ((), ())),
                                 preferred_element_type=F32) * dinv[...]


def _out_body(ta, tb, z, dinv, b, o_ref):
    o = dinv[...] * (ta[...] + tb[...] + z[...])
    o_ref[...] = o[:, :2] + b[...]


def _rb(w):
    return pl.BlockSpec((BLK, w), lambda i: (i, 0))


def _full(shape):
    return pl.BlockSpec(shape, lambda i: tuple(0 for _ in shape))


def kernel(x, edge_index, W1, b1, g1, be1, W2, b2, g2, be2, W3, b3):
    ei = edge_index.astype(jnp.int32)
    srcm = ei[0].reshape(ER, 128)
    dstm = ei[1].reshape(ER, 128)

    zerosNP = jnp.zeros((NP,), F32)
    zeros128 = jnp.zeros((128, 128), F32)

    degp = _sc_deg(dstm, zerosNP)

    deg1d = pl.pallas_call(
        _degred_body,
        grid=(1,),
        in_specs=[_full((32, NP))],
        out_specs=_full((NP,)),
        out_shape=jax.ShapeDtypeStruct((NP,), F32),
    )(degp)
    deg_col = deg1d.reshape(NP, 1)[:N]

    dinv, zs1a, zs1b = pl.pallas_call(
        _pre_body,
        grid=(GRID,),
        in_specs=[_rb(1), _rb(256)],
        out_specs=[_rb(1), _rb(128), _rb(128)],
        out_shape=[jax.ShapeDtypeStruct((N, 1), F32),
                   jax.ShapeDtypeStruct((N, 128), F32),
                   jax.ShapeDtypeStruct((N, 128), F32)],
    )(deg_col, x)

    t1a, t1b = _sc_prop128(zs1a, zs1b, srcm, dstm, zeros128)

    y1, sums1 = pl.pallas_call(
        _l1_body,
        grid=(GRID,),
        in_specs=[_rb(128), _rb(128), _rb(128), _rb(128), _rb(1),
                  _full((256, 512)), _full((1, 512))],
        out_specs=[_rb(512), _full((1, 1024))],
        out_shape=[jax.ShapeDtypeStruct((N, 512), F32),
                   jax.ShapeDtypeStruct((1, 1024), F32)],
    )(t1a, t1b, zs1a, zs1b, dinv, W1, b1.reshape(1, 512))

    zc = pl.pallas_call(
        _bn_mm_body,
        grid=(GRID,),
        in_specs=[_rb(512), _full((1, 1024)), _full((1, 512)),
                  _full((1, 512)), _full((512, 512)), _rb(1)],
        out_specs=[_rb(128)] * 4,
        out_shape=[jax.ShapeDtypeStruct((N, 128), F32)] * 4,
    )(y1, sums1, g1.reshape(1, 512), be1.reshape(1, 512), W2, dinv)

    t2c0, t2c1, t2c2, t2c3 = _sc_prop128x2(
        zc[0], zc[1], zc[2], zc[3], srcm, dstm, zeros128)

    v2, sums2 = pl.pallas_call(
        _l2_body,
        grid=(GRID,),
        in_specs=[_rb(128)] * 4 + [_rb(128)] * 4 + [_rb(1), _full((1, 512))],
        out_specs=[_rb(512), _full((1, 1024))],
        out_shape=[jax.ShapeDtypeStruct((N, 512), F32),
                   jax.ShapeDtypeStruct((1, 1024), F32)],
    )(t2c0, t2c1, t2c2, t2c3, zc[0], zc[1], zc[2], zc[3], dinv,
      b2.reshape(1, 512))

    W3p = jnp.pad(W3, ((0, 0), (0, 126)))
    zs3p = pl.pallas_call(
        _bn_mm128_body,
        grid=(GRID,),
        in_specs=[_rb(512), _full((1, 1024)), _full((1, 512)),
                  _full((1, 512)), _full((512, 128)), _rb(1)],
        out_specs=_rb(128),
        out_shape=jax.ShapeDtypeStruct((N, 128), F32),
    )(v2, sums2, g2.reshape(1, 512), be2.reshape(1, 512), W3p, dinv)

    t3p = _sc_prop128_split(zs3p, srcm, dstm, zeros128)

    out = pl.pallas_call(
        _out_body,
        grid=(GRID,),
        in_specs=[_rb(128), _rb(128), _rb(128), _rb(1), _full((1, 2))],
        out_specs=_rb(2),
        out_shape=jax.ShapeDtypeStruct((N, 2), F32),
    )(t3p[0], t3p[1], zs3p, dinv, b3.reshape(1, 2))
    return out


# speedup vs baseline: 1.2486x; 1.0148x over previous
import functools

import jax
import jax.numpy as jnp
from jax import lax
from jax.experimental import pallas as pl
from jax.experimental.pallas import tpu as pltpu
from jax.experimental.pallas import tpu_sc as plsc

N = 10000
NP = 10240
E = 160000
ER = 1250
EPS = 1e-5
BLK = 2000
GRID = N // BLK

_MESH = plsc.VectorSubcoreMesh(
    core_axis_name="c", subcore_axis_name="s", num_cores=2, num_subcores=16)

F32 = jnp.float32


@functools.partial(
    pl.kernel,
    out_type=jax.ShapeDtypeStruct((32, NP), F32),
    mesh=_MESH,
    scratch_types=[
        pltpu.VMEM((NP,), F32),
        pltpu.VMEM((128,), jnp.int32),
        pltpu.SemaphoreType.DMA,
    ],
    compiler_params=pltpu.CompilerParams(needs_layout_passes=False),
)
def _sc_deg(dstm, zeros_in, outp, hist, drow, sem):
    c = lax.axis_index("c")
    s = lax.axis_index("s")
    wid = c * 16 + s
    pltpu.sync_copy(zeros_in, hist)
    nr = jnp.where(wid < 2, 40, 39)

    ones = jnp.full((16,), 1.0, F32)

    def eb(k, carry):
        row = wid + 32 * k
        pltpu.sync_copy(dstm.at[row], drow)
        for j in range(8):
            idx = drow[pl.ds(16 * j, 16)]
            plsc.addupdate_scatter(hist, [idx], ones)
        return carry

    lax.fori_loop(0, nr, eb, 0)
    pltpu.sync_copy(hist, outp.at[wid])


def _prop_pipeline(c, s, zs_by_core, accum, sbuf, dbuf, rows, sg, ss,
                   nfull):
    def g_start(i, b):
        @pl.when(c == 0)
        def _():
            for h in range(2):
                pltpu.make_async_copy(
                    zs_by_core[0].at[sbuf.at[i, pl.ds(64 * h, 64)]],
                    rows[b].at[pl.ds(64 * h, 64), :], sg[b]).start()

        @pl.when(c == 1)
        def _():
            for h in range(2):
                pltpu.make_async_copy(
                    zs_by_core[1].at[sbuf.at[i, pl.ds(64 * h, 64)]],
                    rows[b].at[pl.ds(64 * h, 64), :], sg[b]).start()

    def g_wait(b):
        @pl.when(c == 0)
        def _():
            pltpu.make_async_copy(
                zs_by_core[0].at[sbuf.at[0]], rows[b], sg[b]).wait()

        @pl.when(c == 1)
        def _():
            pltpu.make_async_copy(
                zs_by_core[1].at[sbuf.at[0]], rows[b], sg[b]).wait()

    def s_start(i, b):
        pltpu.make_async_copy(
            rows[b], accum.at[dbuf.at[i]], ss[b]).start(add=True)

    def s_wait(b):
        pltpu.make_async_copy(
            rows[b], accum.at[dbuf.at[0]], ss[b]).wait()

    g_start(0, 0)
    g_start(1, 1)

    def outer(k, carry):
        for b in range(2):
            i = 2 * k + b
            g_wait(b)
            s_start(i, b)
            s_wait(b)

            @pl.when(i + 2 < nfull)
            def _():
                g_start(i + 2, b)
        return carry

    lax.fori_loop(0, nfull // 2, outer, 0)


def _stage(srcm, dstm, sbuf, dbuf, base, n):
    base = pl.multiple_of(base, 8)
    pltpu.sync_copy(srcm.at[pl.ds(base, n), :], sbuf.at[pl.ds(0, n), :])
    pltpu.sync_copy(dstm.at[pl.ds(base, n), :], dbuf.at[pl.ds(0, n), :])


def _prop_chunk(c, s, zs0, zs1, t0, t1, srcm, dstm, accum, sbuf, dbuf,
                rows, sg, ss, zeros_in):
    rows0 = rows[0]
    pltpu.sync_copy(zeros_in, rows0)
    for j in range(5):
        pltpu.sync_copy(rows0, accum.at[pl.ds(s * 640 + j * 128, 128), :])
    plsc.subcore_barrier()
    baseA = jnp.where(s < 12, 80 * s, 960 + 72 * (s - 12))
    _stage(srcm, dstm, sbuf, dbuf, baseA, 40)
    _prop_pipeline(c, s, [zs0, zs1], accum, sbuf, dbuf, rows, sg, ss, 40)

    @pl.when(s < 12)
    def _():
        _stage(srcm, dstm, sbuf, dbuf, baseA + 40, 40)
        _prop_pipeline(c, s, [zs0, zs1], accum, sbuf, dbuf, rows, sg, ss, 40)

    @pl.when(s >= 12)
    def _():
        _stage(srcm, dstm, sbuf, dbuf, baseA + 40, 32)
        _prop_pipeline(c, s, [zs0, zs1], accum, sbuf, dbuf, rows, sg, ss, 32)

    @pl.when(s >= 14)
    def _():
        pltpu.sync_copy(srcm.at[1248 + (s - 14)], sbuf.at[0])
        pltpu.sync_copy(dstm.at[1248 + (s - 14)], dbuf.at[0])

        @pl.when(c == 0)
        def _():
            pltpu.async_copy(zs0.at[sbuf.at[0]], rows0, sg[0]).wait()

        @pl.when(c == 1)
        def _():
            pltpu.async_copy(zs1.at[sbuf.at[0]], rows0, sg[0]).wait()

        pltpu.sync_copy(rows0, accum.at[dbuf.at[0]], add=True)

    plsc.subcore_barrier()
    for j in range(5):
        pltpu.sync_copy(accum.at[pl.ds(s * 640 + j * 128, 128), :], rows0)

        @pl.when(c == 0)
        def _():
            pltpu.sync_copy(rows0, t0.at[pl.ds(s * 640 + j * 128, 128), :])

        @pl.when(c == 1)
        def _():
            pltpu.sync_copy(rows0, t1.at[pl.ds(s * 640 + j * 128, 128), :])


def _make_prop(nchunks):
    @functools.partial(
        pl.kernel,
        out_type=tuple(jax.ShapeDtypeStruct((NP, 128), F32)
                       for _ in range(2 * nchunks)),
        mesh=_MESH,
        scratch_types=[
            pltpu.VMEM_SHARED((NP, 128), F32),
            pltpu.VMEM((40, 128), jnp.int32),
            pltpu.VMEM((40, 128), jnp.int32),
            pltpu.VMEM((128, 128), F32),
            pltpu.VMEM((128, 128), F32),
            pltpu.SemaphoreType.DMA,
            pltpu.SemaphoreType.DMA,
            pltpu.SemaphoreType.DMA,
            pltpu.SemaphoreType.DMA,
        ],
    )
    def prop(*refs):
        zs = refs[:2 * nchunks]
        srcm, dstm, zeros_in = refs[2 * nchunks:2 * nchunks + 3]
        ts = refs[2 * nchunks + 3:4 * nchunks + 3]
        accum, sbuf, dbuf, r0, r1, sg0, sg1, ss0, ss1 = \
            refs[4 * nchunks + 3:]
        c = lax.axis_index("c")
        s = lax.axis_index("s")
        for ch in range(nchunks):
            _prop_chunk(c, s, zs[2 * ch], zs[2 * ch + 1],
                        ts[2 * ch], ts[2 * ch + 1], srcm, dstm,
                        accum, sbuf, dbuf, [r0, r1],
                        [sg0, sg1], [ss0, ss1], zeros_in)

    return prop


_sc_prop128 = _make_prop(1)
_sc_prop128x2 = _make_prop(2)


@functools.partial(
    pl.kernel,
    out_type=jax.ShapeDtypeStruct((2, NP, 128), F32),
    mesh=_MESH,
    scratch_types=[
        pltpu.VMEM_SHARED((NP, 128), F32),
        pltpu.VMEM((40, 128), jnp.int32),
        pltpu.VMEM((40, 128), jnp.int32),
        pltpu.VMEM((128, 128), F32),
        pltpu.VMEM((128, 128), F32),
        pltpu.SemaphoreType.DMA,
        pltpu.SemaphoreType.DMA,
        pltpu.SemaphoreType.DMA,
        pltpu.SemaphoreType.DMA,
    ],
)
def _sc_prop128_split(zsp, srcm, dstm, zeros_in, outp,
                      accum, sbuf, dbuf, rows0, rows1, sg0, sg1, ss0, ss1):
    c = lax.axis_index("c")
    s = lax.axis_index("s")
    rows = [rows0, rows1]
    pltpu.sync_copy(zeros_in, rows0)
    for j in range(5):
        pltpu.sync_copy(rows0, accum.at[pl.ds(s * 640 + j * 128, 128), :])
    plsc.subcore_barrier()
    baseA = 624 * c + jnp.where(s < 14, 40 * s, 560 + 32 * (s - 14))

    @pl.when(s < 14)
    def _():
        _stage(srcm, dstm, sbuf, dbuf, baseA, 40)
        _prop_pipeline(c, s, [zsp, zsp], accum, sbuf, dbuf, rows,
                       [sg0, sg1], [ss0, ss1], 40)

    @pl.when(s >= 14)
    def _():
        _stage(srcm, dstm, sbuf, dbuf, baseA, 32)
        _prop_pipeline(c, s, [zsp, zsp], accum, sbuf, dbuf, rows,
                       [sg0, sg1], [ss0, ss1], 32)

    @pl.when(s == 0)
    def _():
        pltpu.sync_copy(srcm.at[1248 + c], sbuf.at[0])
        pltpu.sync_copy(dstm.at[1248 + c], dbuf.at[0])
        pltpu.async_copy(zsp.at[sbuf.at[0]], rows0, sg0).wait()
        pltpu.sync_copy(rows0, accum.at[dbuf.at[0]], add=True)

    plsc.subcore_barrier()
    for j in range(5):
        pltpu.sync_copy(accum.at[pl.ds(s * 640 + j * 128, 128), :], rows0)
        pltpu.sync_copy(rows0, outp.at[c, pl.ds(s * 640 + j * 128, 128), :])


def _pre_body(degp, x, dinv, za, zb):
    ones = jnp.ones((32, 1), F32)
    deg = 1.0 + lax.dot_general(degp[...], ones, (((1,), (0,)), ((), ())),
                                preferred_element_type=F32)
    dv = lax.rsqrt(deg)
    dinv[...] = dv
    zs = x[...] * dv
    za[...] = zs[:, :128]
    zb[...] = zs[:, 128:]


def _l1bn1_body(t1a, t1b, za, zb, dinv, w1, b1, g, be, w2,
                z0, z1, z2, z3, y_scr, sums):
    i = pl.program_id(0)
    im = i % GRID

    @pl.when(i < GRID)
    def _():
        u = dinv[...] * jnp.concatenate(
            [t1a[...] + za[...], t1b[...] + zb[...]], axis=1)
        y = lax.dot_general(u, w1[...], (((1,), (0,)), ((), ())),
                            preferred_element_type=F32) + b1[...]
        y_scr[pl.ds(im * BLK, BLK), :] = y

        @pl.when(i == 0)
        def _():
            sums[...] = jnp.zeros_like(sums)

        sums[...] += jnp.concatenate(
            [jnp.sum(y, axis=0, keepdims=True),
             jnp.sum(y * y, axis=0, keepdims=True)], axis=1)

    @pl.when(i >= GRID)
    def _():
        mu = sums[0:1, :512] * (1.0 / N)
        var = sums[0:1, 512:] * (1.0 / N) - mu * mu
        y = y_scr[pl.ds(im * BLK, BLK), :]
        h = jnp.maximum((y - mu) * lax.rsqrt(var + EPS) * g[...] + be[...],
                        0.0)
        z = lax.dot_general(h, w2[...], (((1,), (0,)), ((), ())),
                            preferred_element_type=F32) * dinv[...]
        z0[...] = z[:, 0:128]
        z1[...] = z[:, 128:256]
        z2[...] = z[:, 256:384]
        z3[...] = z[:, 384:512]


def _l2bn2_body(t0, t1, t2, t3, z0, z1, z2, z3, dinv, b2, g, be, w3,
                zp_ref, v_scr, sums):
    i = pl.program_id(0)
    im = i % GRID

    @pl.when(i < GRID)
    def _():
        v = dinv[...] * jnp.concatenate(
            [t0[...] + z0[...], t1[...] + z1[...],
             t2[...] + z2[...], t3[...] + z3[...]], axis=1) + b2[...]
        v_scr[pl.ds(im * BLK, BLK), :] = v

        @pl.when(i == 0)
        def _():
            sums[...] = jnp.zeros_like(sums)

        sums[...] += jnp.concatenate(
            [jnp.sum(v, axis=0, keepdims=True),
             jnp.sum(v * v, axis=0, keepdims=True)], axis=1)

    @pl.when(i >= GRID)
    def _():
        mu = sums[0:1, :512] * (1.0 / N)
        var = sums[0:1, 512:] * (1.0 / N) - mu * mu
        v = v_scr[pl.ds(im * BLK, BLK), :]
        h = jnp.maximum((v - mu) * lax.rsqrt(var + EPS) * g[...] + be[...],
                        0.0)
        zp_ref[...] = lax.dot_general(h, w3[...], (((1,), (0,)), ((), ())),
                                      preferred_element_type=F32) * dinv[...]


def _out_body(ta, tb, z, dinv, b, o_ref):
    o = dinv[...] * (ta[...] + tb[...] + z[...])
    o_ref[...] = o[:, :2] + b[...]


def _rb(w):
    return pl.BlockSpec((BLK, w), lambda i: (i, 0))


def _full(shape):
    return pl.BlockSpec(shape, lambda i: tuple(0 for _ in shape))


def kernel(x, edge_index, W1, b1, g1, be1, W2, b2, g2, be2, W3, b3):
    ei = edge_index.astype(jnp.int32)
    srcm = ei[0].reshape(ER, 128)
    dstm = ei[1].reshape(ER, 128)

    zerosNP = jnp.zeros((NP,), F32)
    zeros128 = jnp.zeros((128, 128), F32)

    degp = _sc_deg(dstm, zerosNP)

    dinv, zs1a, zs1b = pl.pallas_call(
        _pre_body,
        grid=(GRID,),
        in_specs=[pl.BlockSpec((BLK, 32), lambda i: (i, 0)), _rb(256)],
        out_specs=[_rb(1), _rb(128), _rb(128)],
        out_shape=[jax.ShapeDtypeStruct((N, 1), F32),
                   jax.ShapeDtypeStruct((N, 128), F32),
                   jax.ShapeDtypeStruct((N, 128), F32)],
    )(degp.T, x)

    t1a, t1b = _sc_prop128(zs1a, zs1b, srcm, dstm, zeros128)

    def _rb2(w):
        return pl.BlockSpec((BLK, w), lambda i: (i % GRID, 0))

    def _full2(shape):
        return pl.BlockSpec(shape, lambda i: tuple(0 for _ in shape))

    zc = pl.pallas_call(
        _l1bn1_body,
        grid=(2 * GRID,),
        in_specs=[_rb2(128), _rb2(128), _rb2(128), _rb2(128), _rb2(1),
                  _full2((256, 512)), _full2((1, 512)), _full2((1, 512)),
                  _full2((1, 512)), _full2((512, 512))],
        out_specs=[_rb2(128)] * 4,
        out_shape=[jax.ShapeDtypeStruct((N, 128), F32)] * 4,
        scratch_shapes=[pltpu.VMEM((N, 512), F32), pltpu.VMEM((1, 1024), F32)],
    )(t1a, t1b, zs1a, zs1b, dinv, W1, b1.reshape(1, 512),
      g1.reshape(1, 512), be1.reshape(1, 512), W2)

    t2c0, t2c1, t2c2, t2c3 = _sc_prop128x2(
        zc[0], zc[1], zc[2], zc[3], srcm, dstm, zeros128)

    W3p = jnp.pad(W3, ((0, 0), (0, 126)))
    zs3p = pl.pallas_call(
        _l2bn2_body,
        grid=(2 * GRID,),
        in_specs=[_rb2(128)] * 4 + [_rb2(128)] * 4 +
                 [_rb2(1), _full2((1, 512)), _full2((1, 512)),
                  _full2((1, 512)), _full2((512, 128))],
        out_specs=_rb2(128),
        out_shape=jax.ShapeDtypeStruct((N, 128), F32),
        scratch_shapes=[pltpu.VMEM((N, 512), F32), pltpu.VMEM((1, 1024), F32)],
    )(t2c0, t2c1, t2c2, t2c3, zc[0], zc[1], zc[2], zc[3], dinv,
      b2.reshape(1, 512), g2.reshape(1, 512), be2.reshape(1, 512), W3p)

    t3p = _sc_prop128_split(zs3p, srcm, dstm, zeros128)

    out = pl.pallas_call(
        _out_body,
        grid=(GRID,),
        in_specs=[_rb(128), _rb(128), _rb(128), _rb(1), _full((1, 2))],
        out_specs=_rb(2),
        out_shape=jax.ShapeDtypeStruct((N, 2), F32),
    )(t3p[0], t3p[1], zs3p, dinv, b3.reshape(1, 2))
    return out


# speedup vs baseline: 1.2905x; 1.0335x over previous
import functools

import jax
import jax.numpy as jnp
from jax import lax
from jax.experimental import pallas as pl
from jax.experimental.pallas import tpu as pltpu
from jax.experimental.pallas import tpu_sc as plsc

N = 10000
NP = 10240
E = 160000
ER = 1250
EPS = 1e-5
BLK = 2000
GRID = N // BLK

_MESH = plsc.VectorSubcoreMesh(
    core_axis_name="c", subcore_axis_name="s", num_cores=2, num_subcores=16)

F32 = jnp.float32


@functools.partial(
    pl.kernel,
    out_type=jax.ShapeDtypeStruct((32, NP), F32),
    mesh=_MESH,
    scratch_types=[
        pltpu.VMEM((NP,), F32),
        pltpu.VMEM((40, 128), jnp.int32),
        pltpu.SemaphoreType.DMA,
    ],
    compiler_params=pltpu.CompilerParams(needs_layout_passes=False),
)
def _sc_deg(dstm, zeros_in, outp, hist, dstg, sem):
    c = lax.axis_index("c")
    s = lax.axis_index("s")
    wid = c * 16 + s
    pltpu.sync_copy(zeros_in, hist)
    nr = jnp.where(wid < 31, 40, 10)
    base = pl.multiple_of(40 * wid, 8)

    @pl.when(wid < 31)
    def _():
        pltpu.sync_copy(dstm.at[pl.ds(base, 40), :], dstg)

    @pl.when(wid == 31)
    def _():
        pltpu.sync_copy(dstm.at[pl.ds(base, 8), :], dstg.at[pl.ds(0, 8), :])
        pltpu.sync_copy(dstm.at[1248], dstg.at[8])
        pltpu.sync_copy(dstm.at[1249], dstg.at[9])

    ones = jnp.full((16,), 1.0, F32)

    def eb(k, carry):
        for j in range(8):
            idx = dstg[k, pl.ds(16 * j, 16)]
            plsc.addupdate_scatter(hist, [idx], ones)
        return carry

    lax.fori_loop(0, nr, eb, 0)
    pltpu.sync_copy(hist, outp.at[wid])


def _prop_pipeline(c, s, zs_by_core, accum, sbuf, dbuf, rows, sg, ss,
                   nfull):
    def g_start(i, b):
        @pl.when(c == 0)
        def _():
            for h in range(2):
                pltpu.make_async_copy(
                    zs_by_core[0].at[sbuf.at[i, pl.ds(64 * h, 64)]],
                    rows[b].at[pl.ds(64 * h, 64), :], sg[b]).start()

        @pl.when(c == 1)
        def _():
            for h in range(2):
                pltpu.make_async_copy(
                    zs_by_core[1].at[sbuf.at[i, pl.ds(64 * h, 64)]],
                    rows[b].at[pl.ds(64 * h, 64), :], sg[b]).start()

    def g_wait(b):
        @pl.when(c == 0)
        def _():
            pltpu.make_async_copy(
                zs_by_core[0].at[sbuf.at[0]], rows[b], sg[b]).wait()

        @pl.when(c == 1)
        def _():
            pltpu.make_async_copy(
                zs_by_core[1].at[sbuf.at[0]], rows[b], sg[b]).wait()

    def s_start(i, b):
        pltpu.make_async_copy(
            rows[b], accum.at[dbuf.at[i]], ss[b]).start(add=True)

    def s_wait(b):
        pltpu.make_async_copy(
            rows[b], accum.at[dbuf.at[0]], ss[b]).wait()

    g_start(0, 0)
    g_start(1, 1)

    def outer(k, carry):
        for b in range(2):
            i = 2 * k + b
            g_wait(b)
            s_start(i, b)
            s_wait(b)

            @pl.when(i + 2 < nfull)
            def _():
                g_start(i + 2, b)
        return carry

    lax.fori_loop(0, nfull // 2, outer, 0)


def _stage(srcm, dstm, sbuf, dbuf, base, n):
    base = pl.multiple_of(base, 8)
    pltpu.sync_copy(srcm.at[pl.ds(base, n), :], sbuf.at[pl.ds(0, n), :])
    pltpu.sync_copy(dstm.at[pl.ds(base, n), :], dbuf.at[pl.ds(0, n), :])


def _prop_chunk(c, s, zs0, zs1, t0, t1, srcm, dstm, accum, sbuf, dbuf,
                rows, sg, ss, zeros_in):
    rows0 = rows[0]
    pltpu.sync_copy(zeros_in, rows0)
    for j in range(5):
        pltpu.sync_copy(rows0, accum.at[pl.ds(s * 640 + j * 128, 128), :])
    plsc.subcore_barrier()
    baseA = jnp.where(s < 12, 80 * s, 960 + 72 * (s - 12))
    _stage(srcm, dstm, sbuf, dbuf, baseA, 40)
    _prop_pipeline(c, s, [zs0, zs1], accum, sbuf, dbuf, rows, sg, ss, 40)

    @pl.when(s < 12)
    def _():
        _stage(srcm, dstm, sbuf, dbuf, baseA + 40, 40)
        _prop_pipeline(c, s, [zs0, zs1], accum, sbuf, dbuf, rows, sg, ss, 40)

    @pl.when(s >= 12)
    def _():
        _stage(srcm, dstm, sbuf, dbuf, baseA + 40, 32)
        _prop_pipeline(c, s, [zs0, zs1], accum, sbuf, dbuf, rows, sg, ss, 32)

    @pl.when(s >= 14)
    def _():
        pltpu.sync_copy(srcm.at[1248 + (s - 14)], sbuf.at[0])
        pltpu.sync_copy(dstm.at[1248 + (s - 14)], dbuf.at[0])

        @pl.when(c == 0)
        def _():
            pltpu.async_copy(zs0.at[sbuf.at[0]], rows0, sg[0]).wait()

        @pl.when(c == 1)
        def _():
            pltpu.async_copy(zs1.at[sbuf.at[0]], rows0, sg[0]).wait()

        pltpu.sync_copy(rows0, accum.at[dbuf.at[0]], add=True)

    plsc.subcore_barrier()
    for j in range(5):
        pltpu.sync_copy(accum.at[pl.ds(s * 640 + j * 128, 128), :], rows0)

        @pl.when(c == 0)
        def _():
            pltpu.sync_copy(rows0, t0.at[pl.ds(s * 640 + j * 128, 128), :])

        @pl.when(c == 1)
        def _():
            pltpu.sync_copy(rows0, t1.at[pl.ds(s * 640 + j * 128, 128), :])


def _make_prop(nchunks):
    @functools.partial(
        pl.kernel,
        out_type=tuple(jax.ShapeDtypeStruct((NP, 128), F32)
                       for _ in range(2 * nchunks)),
        mesh=_MESH,
        scratch_types=[
            pltpu.VMEM_SHARED((NP, 128), F32),
            pltpu.VMEM((40, 128), jnp.int32),
            pltpu.VMEM((40, 128), jnp.int32),
            pltpu.VMEM((128, 128), F32),
            pltpu.VMEM((128, 128), F32),
            pltpu.SemaphoreType.DMA,
            pltpu.SemaphoreType.DMA,
            pltpu.SemaphoreType.DMA,
            pltpu.SemaphoreType.DMA,
        ],
    )
    def prop(*refs):
        zs = refs[:2 * nchunks]
        srcm, dstm, zeros_in = refs[2 * nchunks:2 * nchunks + 3]
        ts = refs[2 * nchunks + 3:4 * nchunks + 3]
        accum, sbuf, dbuf, r0, r1, sg0, sg1, ss0, ss1 = \
            refs[4 * nchunks + 3:]
        c = lax.axis_index("c")
        s = lax.axis_index("s")
        for ch in range(nchunks):
            _prop_chunk(c, s, zs[2 * ch], zs[2 * ch + 1],
                        ts[2 * ch], ts[2 * ch + 1], srcm, dstm,
                        accum, sbuf, dbuf, [r0, r1],
                        [sg0, sg1], [ss0, ss1], zeros_in)

    return prop


_sc_prop128 = _make_prop(1)
_sc_prop128x2 = _make_prop(2)


@functools.partial(
    pl.kernel,
    out_type=jax.ShapeDtypeStruct((2, NP, 128), F32),
    mesh=_MESH,
    scratch_types=[
        pltpu.VMEM_SHARED((NP, 128), F32),
        pltpu.VMEM((40, 128), jnp.int32),
        pltpu.VMEM((40, 128), jnp.int32),
        pltpu.VMEM((128, 128), F32),
        pltpu.VMEM((128, 128), F32),
        pltpu.SemaphoreType.DMA,
        pltpu.SemaphoreType.DMA,
        pltpu.SemaphoreType.DMA,
        pltpu.SemaphoreType.DMA,
    ],
)
def _sc_prop128_split(zsp, srcm, dstm, zeros_in, outp,
                      accum, sbuf, dbuf, rows0, rows1, sg0, sg1, ss0, ss1):
    c = lax.axis_index("c")
    s = lax.axis_index("s")
    rows = [rows0, rows1]
    pltpu.sync_copy(zeros_in, rows0)
    for j in range(5):
        pltpu.sync_copy(rows0, accum.at[pl.ds(s * 640 + j * 128, 128), :])
    plsc.subcore_barrier()
    baseA = 624 * c + jnp.where(s < 14, 40 * s, 560 + 32 * (s - 14))

    @pl.when(s < 14)
    def _():
        _stage(srcm, dstm, sbuf, dbuf, baseA, 40)
        _prop_pipeline(c, s, [zsp, zsp], accum, sbuf, dbuf, rows,
                       [sg0, sg1], [ss0, ss1], 40)

    @pl.when(s >= 14)
    def _():
        _stage(srcm, dstm, sbuf, dbuf, baseA, 32)
        _prop_pipeline(c, s, [zsp, zsp], accum, sbuf, dbuf, rows,
                       [sg0, sg1], [ss0, ss1], 32)

    @pl.when(s == 0)
    def _():
        pltpu.sync_copy(srcm.at[1248 + c], sbuf.at[0])
        pltpu.sync_copy(dstm.at[1248 + c], dbuf.at[0])
        pltpu.async_copy(zsp.at[sbuf.at[0]], rows0, sg0).wait()
        pltpu.sync_copy(rows0, accum.at[dbuf.at[0]], add=True)

    plsc.subcore_barrier()
    for j in range(5):
        pltpu.sync_copy(accum.at[pl.ds(s * 640 + j * 128, 128), :], rows0)
        pltpu.sync_copy(rows0, outp.at[c, pl.ds(s * 640 + j * 128, 128), :])


def _pre_body(degp, x, dinv, za, zb):
    ones = jnp.ones((32, 1), F32)
    deg = 1.0 + lax.dot_general(degp[...], ones, (((1,), (0,)), ((), ())),
                                preferred_element_type=F32)
    dv = lax.rsqrt(deg)
    dinv[...] = dv
    zs = x[...] * dv
    za[...] = zs[:, :128]
    zb[...] = zs[:, 128:]


def _l1bn1_body(t1a, t1b, za, zb, dinv, w1, b1, g, be, w2,
                z0, z1, z2, z3, y_scr, sums):
    i = pl.program_id(0)
    im = i % GRID

    @pl.when(i < GRID)
    def _():
        u = dinv[...] * jnp.concatenate(
            [t1a[...] + za[...], t1b[...] + zb[...]], axis=1)
        y = lax.dot_general(u, w1[...], (((1,), (0,)), ((), ())),
                            preferred_element_type=F32) + b1[...]
        y_scr[pl.ds(im * BLK, BLK), :] = y

        @pl.when(i == 0)
        def _():
            sums[...] = jnp.zeros_like(sums)

        sums[...] += jnp.concatenate(
            [jnp.sum(y, axis=0, keepdims=True),
             jnp.sum(y * y, axis=0, keepdims=True)], axis=1)

    @pl.when(i >= GRID)
    def _():
        mu = sums[0:1, :512] * (1.0 / N)
        var = sums[0:1, 512:] * (1.0 / N) - mu * mu
        y = y_scr[pl.ds(im * BLK, BLK), :]
        h = jnp.maximum((y - mu) * lax.rsqrt(var + EPS) * g[...] + be[...],
                        0.0)
        z = lax.dot_general(h, w2[...], (((1,), (0,)), ((), ())),
                            preferred_element_type=F32) * dinv[...]
        z0[...] = z[:, 0:128]
        z1[...] = z[:, 128:256]
        z2[...] = z[:, 256:384]
        z3[...] = z[:, 384:512]


def _l2bn2_body(t0, t1, t2, t3, z0, z1, z2, z3, dinv, b2, g, be, w3,
                zp_ref, v_scr, sums):
    i = pl.program_id(0)
    im = i % GRID

    @pl.when(i < GRID)
    def _():
        v = dinv[...] * jnp.concatenate(
            [t0[...] + z0[...], t1[...] + z1[...],
             t2[...] + z2[...], t3[...] + z3[...]], axis=1) + b2[...]
        v_scr[pl.ds(im * BLK, BLK), :] = v

        @pl.when(i == 0)
        def _():
            sums[...] = jnp.zeros_like(sums)

        sums[...] += jnp.concatenate(
            [jnp.sum(v, axis=0, keepdims=True),
             jnp.sum(v * v, axis=0, keepdims=True)], axis=1)

    @pl.when(i >= GRID)
    def _():
        mu = sums[0:1, :512] * (1.0 / N)
        var = sums[0:1, 512:] * (1.0 / N) - mu * mu
        v = v_scr[pl.ds(im * BLK, BLK), :]
        h = jnp.maximum((v - mu) * lax.rsqrt(var + EPS) * g[...] + be[...],
                        0.0)
        zp_ref[...] = lax.dot_general(h, w3[...], (((1,), (0,)), ((), ())),
                                      preferred_element_type=F32) * dinv[...]


def _out_body(ta, tb, z, dinv, b, o_ref):
    o = dinv[...] * (ta[...] + tb[...] + z[...])
    o_ref[...] = o[:, :2] + b[...]


def _rb(w):
    return pl.BlockSpec((BLK, w), lambda i: (i, 0))


def _full(shape):
    return pl.BlockSpec(shape, lambda i: tuple(0 for _ in shape))


def kernel(x, edge_index, W1, b1, g1, be1, W2, b2, g2, be2, W3, b3):
    ei = edge_index.astype(jnp.int32)
    srcm = ei[0].reshape(ER, 128)
    dstm = ei[1].reshape(ER, 128)

    zerosNP = jnp.zeros((NP,), F32)
    zeros128 = jnp.zeros((128, 128), F32)

    degp = _sc_deg(dstm, zerosNP)

    dinv, zs1a, zs1b = pl.pallas_call(
        _pre_body,
        grid=(GRID,),
        in_specs=[pl.BlockSpec((BLK, 32), lambda i: (i, 0)), _rb(256)],
        out_specs=[_rb(1), _rb(128), _rb(128)],
        out_shape=[jax.ShapeDtypeStruct((N, 1), F32),
                   jax.ShapeDtypeStruct((N, 128), F32),
                   jax.ShapeDtypeStruct((N, 128), F32)],
    )(degp.T, x)

    t1a, t1b = _sc_prop128(zs1a, zs1b, srcm, dstm, zeros128)

    def _rb2(w):
        return pl.BlockSpec((BLK, w), lambda i: (i % GRID, 0))

    def _full2(shape):
        return pl.BlockSpec(shape, lambda i: tuple(0 for _ in shape))

    zc = pl.pallas_call(
        _l1bn1_body,
        grid=(2 * GRID,),
        in_specs=[_rb2(128), _rb2(128), _rb2(128), _rb2(128), _rb2(1),
                  _full2((256, 512)), _full2((1, 512)), _full2((1, 512)),
                  _full2((1, 512)), _full2((512, 512))],
        out_specs=[_rb2(128)] * 4,
        out_shape=[jax.ShapeDtypeStruct((N, 128), F32)] * 4,
        scratch_shapes=[pltpu.VMEM((N, 512), F32), pltpu.VMEM((1, 1024), F32)],
    )(t1a, t1b, zs1a, zs1b, dinv, W1, b1.reshape(1, 512),
      g1.reshape(1, 512), be1.reshape(1, 512), W2)

    t2c0, t2c1, t2c2, t2c3 = _sc_prop128x2(
        zc[0], zc[1], zc[2], zc[3], srcm, dstm, zeros128)

    W3p = jnp.pad(W3, ((0, 0), (0, 126)))
    zs3p = pl.pallas_call(
        _l2bn2_body,
        grid=(2 * GRID,),
        in_specs=[_rb2(128)] * 4 + [_rb2(128)] * 4 +
                 [_rb2(1), _full2((1, 512)), _full2((1, 512)),
                  _full2((1, 512)), _full2((512, 128))],
        out_specs=_rb2(128),
        out_shape=jax.ShapeDtypeStruct((N, 128), F32),
        scratch_shapes=[pltpu.VMEM((N, 512), F32), pltpu.VMEM((1, 1024), F32)],
    )(t2c0, t2c1, t2c2, t2c3, zc[0], zc[1], zc[2], zc[3], dinv,
      b2.reshape(1, 512), g2.reshape(1, 512), be2.reshape(1, 512), W3p)

    t3p = _sc_prop128_split(zs3p, srcm, dstm, zeros128)

    out = pl.pallas_call(
        _out_body,
        grid=(GRID,),
        in_specs=[_rb(128), _rb(128), _rb(128), _rb(1), _full((1, 2))],
        out_specs=_rb(2),
        out_shape=jax.ShapeDtypeStruct((N, 2), F32),
    )(t3p[0], t3p[1], zs3p, dinv, b3.reshape(1, 2))
    return out
